# baseline (speedup 1.0000x reference)
import functools

import jax
import jax.numpy as jnp
from jax import lax
from jax.experimental import pallas as pl
from jax.experimental.pallas import tpu as pltpu
from jax.experimental.pallas import tpu_sc as plsc

_B, _T, _C, _H, _E = 1, 2048, 768, 768, 8

_G = 128
_NB = _T // _G + _E
_TPAD = _NB * _G
_NC, _NS = 2, 16
_NW = _NC * _NS
_CHUNK = _TPAD // _NW


def _gate_body(x_ref, wrt_ref, wg_ref, sc_ref, id_ref):
    xg = x_ref[...]
    g = jnp.dot(xg, wrt_ref[...], preferred_element_type=jnp.float32)
    wgv = wg_ref[...]
    nrm = jnp.sqrt(jnp.sum(wgv * wgv, axis=1, keepdims=True))
    wgr = (1.5 / jnp.maximum(nrm, 1e-12)) * wgv
    nrm2 = jnp.sqrt(jnp.sum(wgr * wgr, axis=1, keepdims=True))
    wgn = wgr / jnp.maximum(nrm2, 1e-4)
    logits = lax.dot_general(g, wgn, (((1,), (1,)), ((), ())),
                             preferred_element_type=jnp.float32)
    m = jnp.max(logits, axis=1, keepdims=True)
    p = jnp.exp(logits - m)
    gates = p / jnp.sum(p, axis=1, keepdims=True)
    mx = jnp.max(gates, axis=1, keepdims=True)
    ii = lax.broadcasted_iota(jnp.int32, (_T, _E), 1)
    cand = jnp.where(gates >= mx, ii, _E)
    id_ref[...] = jnp.min(cand, axis=1, keepdims=True)
    sc_ref[...] = mx


def _gate(x2, w_redT, wg):
    return pl.pallas_call(
        _gate_body,
        out_shape=[jax.ShapeDtypeStruct((_T, 1), jnp.float32),
                   jax.ShapeDtypeStruct((_T, 1), jnp.int32)],
    )(x2, w_redT, wg)


@functools.lru_cache(maxsize=1)
def _sc_kernels():
    mesh = plsc.VectorSubcoreMesh(core_axis_name="c", subcore_axis_name="s",
                                  num_cores=_NC, num_subcores=_NS)

    @functools.partial(
        pl.kernel,
        out_type=[jax.ShapeDtypeStruct((_TPAD, _C), jnp.float32),
                  jax.ShapeDtypeStruct((_TPAD, 128), jnp.float32)],
        mesh=mesh,
        scratch_types=[
            pltpu.VMEM((_CHUNK,), jnp.int32),
            pltpu.VMEM((_CHUNK,), jnp.int32),
            pltpu.VMEM((_CHUNK, _C), jnp.float32),
            pltpu.VMEM((_CHUNK, 128), jnp.float32),
            [pltpu.SemaphoreType.DMA] * 4,
            pltpu.SemaphoreType.DMA,
        ],
    )
    def sc_gather(x_hbm, scx_hbm, gidx_hbm, sgidx_hbm, xs_hbm, ss_hbm,
                  idx_v, sidx_v, rows_v, srows_v, sems, sem2):
        wid = lax.axis_index("s") * _NC + lax.axis_index("c")
        base = wid * _CHUNK
        sub = _CHUNK // 4
        pltpu.sync_copy(gidx_hbm.at[pl.ds(base, _CHUNK)], idx_v)
        pltpu.sync_copy(sgidx_hbm.at[pl.ds(base, _CHUNK)], sidx_v)
        cps = [
            pltpu.async_copy(x_hbm.at[idx_v.at[pl.ds(j * sub, sub)]],
                             rows_v.at[pl.ds(j * sub, sub)], sems[j])
            for j in range(4)
        ]
        cp2 = pltpu.async_copy(scx_hbm.at[sidx_v], srows_v, sem2)
        for j in range(4):
            cps[j].wait()
            pltpu.sync_copy(rows_v.at[pl.ds(j * sub, sub)],
                            xs_hbm.at[pl.ds(base + j * sub, sub)])
        cp2.wait()
        pltpu.sync_copy(srows_v, ss_hbm.at[pl.ds(base, _CHUNK)])

    @functools.partial(
        pl.kernel,
        out_type=jax.ShapeDtypeStruct((_T + 8, _C), jnp.float32),
        mesh=mesh,
        scratch_types=[
            pltpu.VMEM((_CHUNK,), jnp.int32),
            pltpu.VMEM((_CHUNK, _C), jnp.float32),
            pltpu.SemaphoreType.DMA,
        ],
    )
    def sc_scatter(ys_hbm, sctidx_hbm, out_hbm, idx_v, rows_v, sem):
        wid = lax.axis_index("s") * _NC + lax.axis_index("c")
        base = wid * _CHUNK
        pltpu.sync_copy(sctidx_hbm.at[pl.ds(base, _CHUNK)], idx_v)
        pltpu.sync_copy(ys_hbm.at[pl.ds(base, _CHUNK)], rows_v)
        pltpu.async_copy(rows_v, out_hbm.at[idx_v], sem).wait()

    return sc_gather, sc_scatter


def _sc_gather(x2, scx, gidx, sgidx):
    return _sc_kernels()[0](x2, scx, gidx, sgidx)


def _sc_scatter(ys, sctidx):
    return _sc_kernels()[1](ys, sctidx)


def _ffn_body(eid_ref, xs_ref, w1_ref, b1_ref, w2_ref, b2_ref, ss_ref, out_ref):
    del eid_ref
    xb = xs_ref[...].astype(jnp.bfloat16)
    h = jnp.dot(xb, w1_ref[0].astype(jnp.bfloat16),
                preferred_element_type=jnp.float32) + b1_ref[0]
    z = h * 0.7071067811865476
    a = jnp.abs(z)
    t = 1.0 / (1.0 + 0.3275911 * a)
    poly = t * (0.254829592 + t * (-0.284496736 + t * (1.421413741
               + t * (-1.453152027 + t * 1.061405429))))
    erf_a = 1.0 - poly * jnp.exp(-a * a)
    erf_z = jnp.where(z < 0.0, -erf_a, erf_a)
    h = 0.5 * h * (1.0 + erf_z)
    y = jnp.dot(h.astype(jnp.bfloat16), w2_ref[0].astype(jnp.bfloat16),
                preferred_element_type=jnp.float32) + b2_ref[0]
    out_ref[...] = y * ss_ref[...]


def _ffn(blk_eid, xs, W1, b1r, W2, b2r, ss2):
    grid_spec = pltpu.PrefetchScalarGridSpec(
        num_scalar_prefetch=1,
        grid=(_NB,),
        in_specs=[
            pl.BlockSpec((_G, _C), lambda i, eid: (i, 0)),
            pl.BlockSpec((1, _C, _H), lambda i, eid: (eid[i], 0, 0)),
            pl.BlockSpec((1, 1, _H), lambda i, eid: (eid[i], 0, 0)),
            pl.BlockSpec((1, _H, _C), lambda i, eid: (eid[i], 0, 0)),
            pl.BlockSpec((1, 1, _C), lambda i, eid: (eid[i], 0, 0)),
            pl.BlockSpec((_G, 1), lambda i, eid: (i, 0)),
        ],
        out_specs=pl.BlockSpec((_G, _C), lambda i, eid: (i, 0)),
    )
    return pl.pallas_call(
        _ffn_body,
        grid_spec=grid_spec,
        out_shape=jax.ShapeDtypeStruct((_TPAD, _C), jnp.float32),
        compiler_params=pltpu.CompilerParams(
            dimension_semantics=("arbitrary",)),
    )(blk_eid, xs, W1, b1r, W2, b2r, ss2)


def kernel(x, w_red, wg, W1, b1, W2, b2):
    Bx, Tx, Cx = x.shape
    x2 = x.reshape(Tx, Cx)

    scores2, idx2 = _gate(x2, w_red.T, wg)
    idx = idx2[:, 0]

    i32 = jnp.int32
    oh = (idx[:, None] == jnp.arange(_E, dtype=i32)[None, :]).astype(i32)
    pos = jnp.cumsum(oh, axis=0)
    pos_in = jnp.take_along_axis(pos, idx[:, None], axis=1)[:, 0] - 1
    counts = pos[-1]
    ntiles = (counts + _G - 1) // _G
    cumblk = jnp.cumsum(ntiles)
    pad_start = (jnp.concatenate([jnp.zeros((1,), i32), cumblk[:-1]]) * _G)
    ppos = pad_start[idx] + pos_in
    tok = jnp.arange(_T, dtype=i32)
    gidx = jnp.zeros((_TPAD,), i32).at[ppos].set(tok)
    sgidx = jnp.full((_TPAD,), _T, i32).at[ppos].set(tok)
    trash = _T + (jnp.arange(_TPAD, dtype=i32) % 8)
    sctidx = trash.at[ppos].set(tok)
    blk = jnp.arange(_NB, dtype=i32)
    blk_eid = jnp.minimum(
        jnp.searchsorted(cumblk, blk, side="right").astype(i32), _E - 1)

    scx = jnp.broadcast_to(
        jnp.concatenate([scores2, jnp.zeros((16, 1), jnp.float32)]),
        (_T + 16, 128))

    xs, ss = _sc_gather(x2, scx, gidx, sgidx)
    ys = _ffn(blk_eid, xs, W1, b1.reshape(_E, 1, _H), W2,
              b2.reshape(_E, 1, _C), ss[:, :1])
    out_pad = _sc_scatter(ys, sctidx)

    out = out_pad[:_T].reshape(Bx, Tx, Cx)
    return (out, jnp.sum(out))


# speedup vs baseline: 1.6954x; 1.6954x over previous
import functools

import jax
import jax.numpy as jnp
from jax import lax
from jax.experimental import pallas as pl
from jax.experimental.pallas import tpu as pltpu
from jax.experimental.pallas import tpu_sc as plsc

_B, _T, _C, _H, _E = 1, 2048, 768, 768, 8

_G = 128
_NB = _T // _G + _E
_TPAD = _NB * _G
_NC, _NS = 2, 16
_NW = _NC * _NS
_CHUNK = _TPAD // _NW
_TCH = _T // _NW


def _gate_body(x_ref, wrt_ref, wg_ref, sc_ref, id_ref):
    xg = x_ref[...]
    g = jnp.dot(xg, wrt_ref[...], preferred_element_type=jnp.float32)
    wgv = wg_ref[...]
    nrm = jnp.sqrt(jnp.sum(wgv * wgv, axis=1, keepdims=True))
    wgr = (1.5 / jnp.maximum(nrm, 1e-12)) * wgv
    nrm2 = jnp.sqrt(jnp.sum(wgr * wgr, axis=1, keepdims=True))
    wgn = wgr / jnp.maximum(nrm2, 1e-4)
    logits = lax.dot_general(g, wgn, (((1,), (1,)), ((), ())),
                             preferred_element_type=jnp.float32)
    m = jnp.max(logits, axis=1, keepdims=True)
    p = jnp.exp(logits - m)
    gates = p / jnp.sum(p, axis=1, keepdims=True)
    mx = jnp.max(gates, axis=1, keepdims=True)
    ii = lax.broadcasted_iota(jnp.int32, (_T, _E), 1)
    cand = jnp.where(gates >= mx, ii, _E)
    id_ref[...] = jnp.min(cand, axis=1, keepdims=True)
    sc_ref[...] = mx


def _gate(x2, w_redT, wg):
    return pl.pallas_call(
        _gate_body,
        out_shape=[jax.ShapeDtypeStruct((_T, 1), jnp.float32),
                   jax.ShapeDtypeStruct((_T, 1), jnp.int32)],
    )(x2, w_redT, wg)


@functools.lru_cache(maxsize=1)
def _sc_kernels():
    mesh = plsc.VectorSubcoreMesh(core_axis_name="c", subcore_axis_name="s",
                                  num_cores=_NC, num_subcores=_NS)

    @functools.partial(
        pl.kernel,
        out_type=[jax.ShapeDtypeStruct((_TPAD, _C), jnp.float32),
                  jax.ShapeDtypeStruct((_TPAD, 128), jnp.float32)],
        mesh=mesh,
        scratch_types=[
            pltpu.VMEM((_TCH,), jnp.int32),
            pltpu.VMEM((_TCH, _C), jnp.float32),
            pltpu.VMEM((_TCH, 128), jnp.float32),
            pltpu.SemaphoreType.DMA,
            pltpu.SemaphoreType.DMA,
        ],
    )
    def sc_dispatch(x_hbm, scx_hbm, ppos_hbm, xs_hbm, ss_hbm,
                    pos_v, rows_v, srows_v, sem, sem2):
        wid = lax.axis_index("s") * _NC + lax.axis_index("c")
        base = wid * _TCH
        pltpu.sync_copy(ppos_hbm.at[pl.ds(base, _TCH)], pos_v)
        pltpu.sync_copy(x_hbm.at[pl.ds(base, _TCH)], rows_v)
        pltpu.sync_copy(scx_hbm.at[pl.ds(base, _TCH)], srows_v)
        cp1 = pltpu.async_copy(rows_v, xs_hbm.at[pos_v], sem)
        cp2 = pltpu.async_copy(srows_v, ss_hbm.at[pos_v], sem2)
        cp1.wait()
        cp2.wait()

    @functools.partial(
        pl.kernel,
        out_type=jax.ShapeDtypeStruct((_T + 8, _C), jnp.float32),
        mesh=mesh,
        scratch_types=[
            pltpu.VMEM((_CHUNK,), jnp.int32),
            pltpu.VMEM((_CHUNK, _C), jnp.float32),
            pltpu.SemaphoreType.DMA,
        ],
    )
    def sc_scatter(ys_hbm, sctidx_hbm, out_hbm, idx_v, rows_v, sem):
        wid = lax.axis_index("s") * _NC + lax.axis_index("c")
        base = wid * _CHUNK
        pltpu.sync_copy(sctidx_hbm.at[pl.ds(base, _CHUNK)], idx_v)
        pltpu.sync_copy(ys_hbm.at[pl.ds(base, _CHUNK)], rows_v)
        pltpu.async_copy(rows_v, out_hbm.at[idx_v], sem).wait()

    return sc_dispatch, sc_scatter


def _sc_dispatch(x2, scx, ppos):
    return _sc_kernels()[0](x2, scx, ppos)


def _sc_scatter(ys, sctidx):
    return _sc_kernels()[1](ys, sctidx)


def _ffn_body(eid_ref, xs_ref, w1_ref, b1_ref, w2_ref, b2_ref, ss_ref, out_ref):
    del eid_ref
    xb = xs_ref[...].astype(jnp.bfloat16)
    h = jnp.dot(xb, w1_ref[0].astype(jnp.bfloat16),
                preferred_element_type=jnp.float32) + b1_ref[0]
    z = h * 0.7071067811865476
    a = jnp.abs(z)
    t = 1.0 / (1.0 + 0.3275911 * a)
    poly = t * (0.254829592 + t * (-0.284496736 + t * (1.421413741
               + t * (-1.453152027 + t * 1.061405429))))
    erf_a = 1.0 - poly * jnp.exp(-a * a)
    erf_z = jnp.where(z < 0.0, -erf_a, erf_a)
    h = 0.5 * h * (1.0 + erf_z)
    y = jnp.dot(h.astype(jnp.bfloat16), w2_ref[0].astype(jnp.bfloat16),
                preferred_element_type=jnp.float32) + b2_ref[0]
    out_ref[...] = y * ss_ref[...]


def _ffn(blk_eid, xs, W1, b1r, W2, b2r, ss2):
    grid_spec = pltpu.PrefetchScalarGridSpec(
        num_scalar_prefetch=1,
        grid=(_NB,),
        in_specs=[
            pl.BlockSpec((_G, _C), lambda i, eid: (i, 0)),
            pl.BlockSpec((1, _C, _H), lambda i, eid: (eid[i], 0, 0)),
            pl.BlockSpec((1, 1, _H), lambda i, eid: (eid[i], 0, 0)),
            pl.BlockSpec((1, _H, _C), lambda i, eid: (eid[i], 0, 0)),
            pl.BlockSpec((1, 1, _C), lambda i, eid: (eid[i], 0, 0)),
            pl.BlockSpec((_G, 1), lambda i, eid: (i, 0)),
        ],
        out_specs=pl.BlockSpec((_G, _C), lambda i, eid: (i, 0)),
    )
    return pl.pallas_call(
        _ffn_body,
        grid_spec=grid_spec,
        out_shape=jax.ShapeDtypeStruct((_TPAD, _C), jnp.float32),
        compiler_params=pltpu.CompilerParams(
            dimension_semantics=("arbitrary",)),
    )(blk_eid, xs, W1, b1r, W2, b2r, ss2)


def kernel(x, w_red, wg, W1, b1, W2, b2):
    Bx, Tx, Cx = x.shape
    x2 = x.reshape(Tx, Cx)

    scores2, idx2 = _gate(x2, w_red.T, wg)
    idx = idx2[:, 0]

    i32 = jnp.int32
    oh = (idx[:, None] == jnp.arange(_E, dtype=i32)[None, :]).astype(i32)
    pos = jnp.cumsum(oh, axis=0)
    pos_in = jnp.take_along_axis(pos, idx[:, None], axis=1)[:, 0] - 1
    counts = pos[-1]
    ntiles = (counts + _G - 1) // _G
    cumblk = jnp.cumsum(ntiles)
    pad_start = (jnp.concatenate([jnp.zeros((1,), i32), cumblk[:-1]]) * _G)
    ppos = pad_start[idx] + pos_in
    tok = jnp.arange(_T, dtype=i32)
    trash = _T + (jnp.arange(_TPAD, dtype=i32) % 8)
    sctidx = trash.at[ppos].set(tok)
    blk = jnp.arange(_NB, dtype=i32)
    blk_eid = jnp.minimum(
        jnp.searchsorted(cumblk, blk, side="right").astype(i32), _E - 1)

    scx = jnp.broadcast_to(scores2, (_T, 128))

    xs, ss = _sc_dispatch(x2, scx, ppos)
    ys = _ffn(blk_eid, xs, W1, b1.reshape(_E, 1, _H), W2,
              b2.reshape(_E, 1, _C), ss[:, :1])
    out_pad = _sc_scatter(ys, sctidx)

    out = out_pad[:_T].reshape(Bx, Tx, Cx)
    return (out, jnp.sum(out))


# speedup vs baseline: 2.0655x; 1.2183x over previous
import functools

import jax
import jax.numpy as jnp
from jax import lax
from jax.experimental import pallas as pl
from jax.experimental.pallas import tpu as pltpu
from jax.experimental.pallas import tpu_sc as plsc

_B, _T, _C, _H, _E = 1, 2048, 768, 768, 8

_G = 128
_NB = _T // _G + _E
_TPAD = _NB * _G
_NC, _NS = 2, 16
_NW = _NC * _NS
_CHUNK = _TPAD // _NW
_TCH = _T // _NW


def _gate_body(x_ref, wrt_ref, wg_ref, scx_ref, pp_ref, eid_ref):
    xg = x_ref[...]
    g = jnp.dot(xg, wrt_ref[...], preferred_element_type=jnp.float32)
    wgv = wg_ref[...]
    nrm = jnp.sqrt(jnp.sum(wgv * wgv, axis=1, keepdims=True))
    wgr = (1.5 / jnp.maximum(nrm, 1e-12)) * wgv
    nrm2 = jnp.sqrt(jnp.sum(wgr * wgr, axis=1, keepdims=True))
    wgn = wgr / jnp.maximum(nrm2, 1e-4)
    logits = lax.dot_general(g, wgn, (((1,), (1,)), ((), ())),
                             preferred_element_type=jnp.float32)
    m = jnp.max(logits, axis=1, keepdims=True)
    p = jnp.exp(logits - m)
    gates = p / jnp.sum(p, axis=1, keepdims=True)
    mx = jnp.max(gates, axis=1, keepdims=True)
    ii = lax.broadcasted_iota(jnp.int32, (_T, _E), 1)
    cand = jnp.where(gates >= mx, ii, _E)
    idxc = jnp.min(cand, axis=1, keepdims=True)
    scx_ref[...] = jnp.broadcast_to(mx, (_T, 128))

    oh = (idxc == ii).astype(jnp.float32)
    rr = lax.broadcasted_iota(jnp.int32, (_G, _G), 0)
    cc = lax.broadcasted_iota(jnp.int32, (_G, _G), 1)
    tri = (rr >= cc).astype(jnp.float32)
    carry = jnp.zeros((1, _E), jnp.float32)
    for k in range(_T // _G):
        ohk = oh[k * _G:(k + 1) * _G, :]
        csk = jnp.dot(tri, ohk, preferred_element_type=jnp.float32) + carry
        carry = csk[_G - 1:_G, :]
        pp_ref[pl.ds(k * _G, _G), :] = (
            jnp.sum(ohk * csk, axis=1, keepdims=True).astype(jnp.int32))
    counts = carry[0:1, :]
    ntiles = jnp.floor((counts + (_G - 1)) * (1.0 / _G))
    ute = (lax.broadcasted_iota(jnp.int32, (_E, _E), 0)
           <= lax.broadcasted_iota(jnp.int32, (_E, _E), 1)).astype(jnp.float32)
    cumblk = jnp.dot(ntiles, ute, preferred_element_type=jnp.float32)
    pad_start = (cumblk - ntiles) * _G
    base = jnp.sum(oh * pad_start, axis=1, keepdims=True)
    pp_ref[...] = pp_ref[...] + base.astype(jnp.int32) - 1
    bj = lax.broadcasted_iota(jnp.int32, (_NB, _E), 0).astype(jnp.float32)
    cmp = (bj >= jnp.broadcast_to(cumblk, (_NB, _E))).astype(jnp.int32)
    eid_ref[...] = jnp.minimum(jnp.sum(cmp, axis=1, keepdims=True), _E - 1)


def _gate(x2, w_redT, wg):
    return pl.pallas_call(
        _gate_body,
        out_shape=[jax.ShapeDtypeStruct((_T, 128), jnp.float32),
                   jax.ShapeDtypeStruct((_T, 1), jnp.int32),
                   jax.ShapeDtypeStruct((_NB, 1), jnp.int32)],
    )(x2, w_redT, wg)


@functools.lru_cache(maxsize=1)
def _sc_kernels():
    mesh = plsc.VectorSubcoreMesh(core_axis_name="c", subcore_axis_name="s",
                                  num_cores=_NC, num_subcores=_NS)

    @functools.partial(
        pl.kernel,
        out_type=[jax.ShapeDtypeStruct((_TPAD, _C), jnp.float32),
                  jax.ShapeDtypeStruct((_TPAD, 128), jnp.float32)],
        mesh=mesh,
        scratch_types=[
            pltpu.VMEM((_TCH,), jnp.int32),
            pltpu.VMEM((_TCH, _C), jnp.float32),
            pltpu.VMEM((_TCH, 128), jnp.float32),
            pltpu.SemaphoreType.DMA,
            pltpu.SemaphoreType.DMA,
        ],
    )
    def sc_dispatch(x_hbm, scx_hbm, ppos_hbm, xs_hbm, ss_hbm,
                    pos_v, rows_v, srows_v, sem, sem2):
        wid = lax.axis_index("s") * _NC + lax.axis_index("c")
        base = wid * _TCH
        pltpu.sync_copy(ppos_hbm.at[pl.ds(base, _TCH)], pos_v)
        pltpu.sync_copy(x_hbm.at[pl.ds(base, _TCH)], rows_v)
        pltpu.sync_copy(scx_hbm.at[pl.ds(base, _TCH)], srows_v)
        cp1 = pltpu.async_copy(rows_v, xs_hbm.at[pos_v], sem)
        cp2 = pltpu.async_copy(srows_v, ss_hbm.at[pos_v], sem2)
        cp1.wait()
        cp2.wait()

    @functools.partial(
        pl.kernel,
        out_type=jax.ShapeDtypeStruct((_T + 8, _C), jnp.float32),
        mesh=mesh,
        scratch_types=[
            pltpu.VMEM((_CHUNK,), jnp.int32),
            pltpu.VMEM((_CHUNK, _C), jnp.float32),
            pltpu.SemaphoreType.DMA,
        ],
    )
    def sc_scatter(ys_hbm, sctidx_hbm, out_hbm, idx_v, rows_v, sem):
        wid = lax.axis_index("s") * _NC + lax.axis_index("c")
        base = wid * _CHUNK
        pltpu.sync_copy(sctidx_hbm.at[pl.ds(base, _CHUNK)], idx_v)
        pltpu.sync_copy(ys_hbm.at[pl.ds(base, _CHUNK)], rows_v)
        pltpu.async_copy(rows_v, out_hbm.at[idx_v], sem).wait()

    return sc_dispatch, sc_scatter


def _sc_dispatch(x2, scx, ppos):
    return _sc_kernels()[0](x2, scx, ppos)


def _sc_scatter(ys, sctidx):
    return _sc_kernels()[1](ys, sctidx)


def _ffn_body(eid_ref, xs_ref, w1_ref, b1_ref, w2_ref, b2_ref, ss_ref, out_ref):
    del eid_ref
    xb = xs_ref[...].astype(jnp.bfloat16)
    h = jnp.dot(xb, w1_ref[0].astype(jnp.bfloat16),
                preferred_element_type=jnp.float32) + b1_ref[0]
    z = h * 0.7071067811865476
    a = jnp.abs(z)
    t = 1.0 / (1.0 + 0.3275911 * a)
    poly = t * (0.254829592 + t * (-0.284496736 + t * (1.421413741
               + t * (-1.453152027 + t * 1.061405429))))
    erf_a = 1.0 - poly * jnp.exp(-a * a)
    erf_z = jnp.where(z < 0.0, -erf_a, erf_a)
    h = 0.5 * h * (1.0 + erf_z)
    y = jnp.dot(h.astype(jnp.bfloat16), w2_ref[0].astype(jnp.bfloat16),
                preferred_element_type=jnp.float32) + b2_ref[0]
    out_ref[...] = y * ss_ref[...]


def _ffn(blk_eid, xs, W1, b1r, W2, b2r, ss2):
    grid_spec = pltpu.PrefetchScalarGridSpec(
        num_scalar_prefetch=1,
        grid=(_NB,),
        in_specs=[
            pl.BlockSpec((_G, _C), lambda i, eid: (i, 0)),
            pl.BlockSpec((1, _C, _H), lambda i, eid: (eid[i], 0, 0)),
            pl.BlockSpec((1, 1, _H), lambda i, eid: (eid[i], 0, 0)),
            pl.BlockSpec((1, _H, _C), lambda i, eid: (eid[i], 0, 0)),
            pl.BlockSpec((1, 1, _C), lambda i, eid: (eid[i], 0, 0)),
            pl.BlockSpec((_G, 1), lambda i, eid: (i, 0)),
        ],
        out_specs=pl.BlockSpec((_G, _C), lambda i, eid: (i, 0)),
    )
    return pl.pallas_call(
        _ffn_body,
        grid_spec=grid_spec,
        out_shape=jax.ShapeDtypeStruct((_TPAD, _C), jnp.float32),
        compiler_params=pltpu.CompilerParams(
            dimension_semantics=("arbitrary",)),
    )(blk_eid, xs, W1, b1r, W2, b2r, ss2)


def kernel(x, w_red, wg, W1, b1, W2, b2):
    Bx, Tx, Cx = x.shape
    x2 = x.reshape(Tx, Cx)

    scx, ppos2, blk_eid2 = _gate(x2, w_red.T, wg)
    ppos = ppos2[:, 0]

    i32 = jnp.int32
    tok = jnp.arange(_T, dtype=i32)
    trash = _T + (jnp.arange(_TPAD, dtype=i32) % 8)
    sctidx = trash.at[ppos].set(tok)

    xs, ss = _sc_dispatch(x2, scx, ppos)
    ys = _ffn(blk_eid2[:, 0], xs, W1, b1.reshape(_E, 1, _H), W2,
              b2.reshape(_E, 1, _C), ss[:, :1])
    out_pad = _sc_scatter(ys, sctidx)

    out = out_pad[:_T].reshape(Bx, Tx, Cx)
    return (out, jnp.sum(out))
